# baseline (speedup 1.0000x reference)
import functools

import jax
import jax.numpy as jnp
from jax import lax
from jax.experimental import pallas as pl
from jax.experimental.pallas import tpu as pltpu
from jax.experimental.pallas import tpu_sc as plsc

N = 10000
D = 128
E = 320000

NP = 10240
CH = 128
NCHUNK = 2560
EP = NCHUNK * CH
NWORKERS = 32
CPW = NCHUNK // NWORKERS
RPT = NP // 16


def _sc_aggregate(with_counts):
    if with_counts:
        out_type = [jax.ShapeDtypeStruct((2 * NP, D), jnp.float32),
                    jax.ShapeDtypeStruct((2 * NP,), jnp.float32)]
    else:
        out_type = jax.ShapeDtypeStruct((2 * NP, D), jnp.float32)

    scratch = [
        pltpu.VMEM_SHARED((NP, D), jnp.float32),
        pltpu.VMEM_SHARED((NP,), jnp.float32),
        pltpu.VMEM((CH, D), jnp.float32),
        pltpu.VMEM((2, CH), jnp.int32),
        pltpu.VMEM((CH,), jnp.float32),
        pltpu.SemaphoreType.DMA,
        pltpu.SemaphoreType.DMA,
        pltpu.SemaphoreType.DMA,
    ]

    mesh = plsc.VectorSubcoreMesh(core_axis_name="c", subcore_axis_name="s")

    @functools.partial(pl.kernel, mesh=mesh, out_type=out_type,
                       scratch_types=scratch)
    def sc_agg(table_hbm, eidx_hbm, zrows_hbm, zcol_hbm, ones_hbm,
               *refs):
        if with_counts:
            parts_hbm, counts_hbm = refs[0], refs[1]
            scr = refs[2:]
        else:
            parts_hbm = refs[0]
            scr = refs[1:]
        acc_sh, cnt_sh, row, idx, ones_v, gsem, rsem, csem = scr

        cid = lax.axis_index("c")
        sid = lax.axis_index("s")
        base = (cid * 16 + sid) * CPW
        sl = pl.ds(sid * RPT, RPT)

        pltpu.sync_copy(zrows_hbm, acc_sh.at[sl])
        if with_counts:
            pltpu.sync_copy(zcol_hbm, cnt_sh.at[sl])
        pltpu.sync_copy(ones_hbm, ones_v)
        plsc.subcore_barrier()

        def body(i, carry):
            pltpu.sync_copy(eidx_hbm.at[base + i], idx)
            g = pltpu.make_async_copy(table_hbm.at[idx.at[0]], row, gsem)
            g.start()
            g.wait()
            s = pltpu.make_async_copy(row, acc_sh.at[idx.at[1]], rsem)
            s.start(add=True)
            if with_counts:
                c = pltpu.make_async_copy(ones_v, cnt_sh.at[idx.at[1]],
                                          csem)
                c.start(add=True)
                s.wait()
                c.wait()
            else:
                s.wait()
            return carry

        lax.fori_loop(0, CPW, body, 0)
        plsc.subcore_barrier()

        off = cid * NP + sid * RPT
        pltpu.sync_copy(acc_sh.at[sl], parts_hbm.at[pl.ds(off, RPT)])
        if with_counts:
            pltpu.sync_copy(cnt_sh.at[sl], counts_hbm.at[pl.ds(off, RPT)])

    return sc_agg


_sc_agg_counts = _sc_aggregate(True)
_sc_agg_nocounts = _sc_aggregate(False)


R = 512
GRID = NP // R


def _tc1_body(p0, p1, cnt, x, wl, bl, wr, h_out):
    c = cnt[0, :] + cnt[1, :]
    inv = 1.0 / jnp.maximum(c, 1.0)
    mean = (p0[...] + p1[...]) * inv[:, None]
    h = (jnp.dot(mean, wl[...], preferred_element_type=jnp.float32)
         + bl[...]
         + jnp.dot(x[...], wr[...], preferred_element_type=jnp.float32))
    h_out[...] = jnp.maximum(h, 0.0)


def _tc2_body(p0, p1, cnt, h, wl, bl, wr, wlin, blin, out):
    c = cnt[0, :] + cnt[1, :]
    inv = 1.0 / jnp.maximum(c, 1.0)
    mean = (p0[...] + p1[...]) * inv[:, None]
    h2 = (jnp.dot(mean, wl[...], preferred_element_type=jnp.float32)
          + bl[...]
          + jnp.dot(h[...], wr[...], preferred_element_type=jnp.float32))
    h2 = jnp.maximum(h2, 0.0)
    out[...] = (jnp.dot(h2, wlin[...], preferred_element_type=jnp.float32)
                + blin[...])


_row_spec = pl.BlockSpec((R, D), lambda i: (i, 0))
_cnt_spec = pl.BlockSpec((2, R), lambda i: (0, i))
_w_spec = pl.BlockSpec((D, D), lambda i: (0, 0))
_b_spec = pl.BlockSpec((1, D), lambda i: (0, 0))

_tc1 = pl.pallas_call(
    _tc1_body,
    grid=(GRID,),
    in_specs=[_row_spec, _row_spec, _cnt_spec, _row_spec,
              _w_spec, _b_spec, _w_spec],
    out_specs=_row_spec,
    out_shape=jax.ShapeDtypeStruct((NP, D), jnp.float32),
)

_tc2 = pl.pallas_call(
    _tc2_body,
    grid=(GRID,),
    in_specs=[_row_spec, _row_spec, _cnt_spec, _row_spec,
              _w_spec, _b_spec, _w_spec, _w_spec, _b_spec],
    out_specs=_row_spec,
    out_shape=jax.ShapeDtypeStruct((NP, D), jnp.float32),
)


def kernel(x, edge_index, Wl1, bl1, Wr1, Wl2, bl2, Wr2, Wlin, blin):
    src = edge_index[0].astype(jnp.int32)
    dst = edge_index[1].astype(jnp.int32)
    pad = EP - E
    src2d = jnp.concatenate(
        [src, jnp.zeros((pad,), jnp.int32)]).reshape(NCHUNK, CH)
    dst2d = jnp.concatenate(
        [dst, N + (jnp.arange(pad, dtype=jnp.int32) % (NP - N))]
    ).reshape(NCHUNK, CH)
    eidx = jnp.stack([src2d, dst2d], axis=1)

    x_pad = jnp.pad(x, ((0, NP - N), (0, 0)))
    zrows = jnp.zeros((RPT, D), jnp.float32)
    zcol = jnp.zeros((RPT,), jnp.float32)
    ones = jnp.ones((CH,), jnp.float32)

    parts1, counts = _sc_agg_counts(x_pad, eidx, zrows, zcol, ones)
    cnt2 = counts.reshape(2, NP)
    h = _tc1(parts1[:NP], parts1[NP:], cnt2, x_pad,
             Wl1, bl1.reshape(1, D), Wr1)

    parts2 = _sc_agg_nocounts(h, eidx, zrows, zcol, ones)
    out = _tc2(parts2[:NP], parts2[NP:], cnt2, h,
               Wl2, bl2.reshape(1, D), Wr2, Wlin, blin.reshape(1, D))
    return out[:N]


# speedup vs baseline: 1.0621x; 1.0621x over previous
import functools

import jax
import jax.numpy as jnp
from jax import lax
from jax.experimental import pallas as pl
from jax.experimental.pallas import tpu as pltpu
from jax.experimental.pallas import tpu_sc as plsc

N = 10000
D = 128
E = 320000

NP = 10240
CH = 128
NCHUNK = 2560
EP = NCHUNK * CH
NWORKERS = 32
CPW = NCHUNK // NWORKERS
RPT = NP // 16


def _sc_aggregate(with_counts):
    if with_counts:
        out_type = [jax.ShapeDtypeStruct((2 * NP, D), jnp.float32),
                    jax.ShapeDtypeStruct((2 * NP,), jnp.float32)]
    else:
        out_type = jax.ShapeDtypeStruct((2 * NP, D), jnp.float32)

    scratch = [
        pltpu.VMEM_SHARED((NP, D), jnp.float32),
        pltpu.VMEM_SHARED((NP,), jnp.float32),
        pltpu.VMEM((CH, D), jnp.float32),
        pltpu.VMEM((2, CH), jnp.int32),
        pltpu.VMEM((2, CH), jnp.int32),
        pltpu.VMEM((CH,), jnp.float32),
        pltpu.SemaphoreType.DMA,
        pltpu.SemaphoreType.DMA,
        pltpu.SemaphoreType.DMA,
        pltpu.SemaphoreType.DMA,
    ]

    mesh = plsc.VectorSubcoreMesh(core_axis_name="c", subcore_axis_name="s")

    @functools.partial(pl.kernel, mesh=mesh, out_type=out_type,
                       scratch_types=scratch)
    def sc_agg(table_hbm, eidx_hbm, zrows_hbm, zcol_hbm, ones_hbm,
               *refs):
        if with_counts:
            parts_hbm, counts_hbm = refs[0], refs[1]
            scr = refs[2:]
        else:
            parts_hbm = refs[0]
            scr = refs[1:]
        acc_sh, cnt_sh, row, idx0, idx1, ones_v, isem, gsem, rsem, csem = scr

        cid = lax.axis_index("c")
        sid = lax.axis_index("s")
        base = (cid * 16 + sid) * CPW
        sl = pl.ds(sid * RPT, RPT)

        pltpu.sync_copy(zrows_hbm, acc_sh.at[sl])
        if with_counts:
            pltpu.sync_copy(zcol_hbm, cnt_sh.at[sl])
        pltpu.sync_copy(ones_hbm, ones_v)
        plsc.subcore_barrier()

        pltpu.sync_copy(eidx_hbm.at[base], idx0)

        def chunk(i, idx, nxt):
            g = pltpu.make_async_copy(table_hbm.at[idx.at[0]], row, gsem)
            g.start()
            g.wait()
            s = pltpu.make_async_copy(row, acc_sh.at[idx.at[1]], rsem)
            s.start(add=True)
            if with_counts:
                c = pltpu.make_async_copy(ones_v, cnt_sh.at[idx.at[1]],
                                          csem)
                c.start(add=True)
            nx = jnp.minimum(base + i + 1, NCHUNK - 1)
            f = pltpu.make_async_copy(eidx_hbm.at[nx], nxt, isem)
            f.start()
            s.wait()
            if with_counts:
                c.wait()
            f.wait()

        def body(k, carry):
            i = 2 * k
            chunk(i, idx0, idx1)
            chunk(i + 1, idx1, idx0)
            return carry

        lax.fori_loop(0, CPW // 2, body, 0)
        plsc.subcore_barrier()

        off = cid * NP + sid * RPT
        pltpu.sync_copy(acc_sh.at[sl], parts_hbm.at[pl.ds(off, RPT)])
        if with_counts:
            pltpu.sync_copy(cnt_sh.at[sl], counts_hbm.at[pl.ds(off, RPT)])

    return sc_agg


_sc_agg_counts = _sc_aggregate(True)
_sc_agg_nocounts = _sc_aggregate(False)


R = 512
GRID = NP // R


def _tc1_body(p0, p1, cnt, x, wl, bl, wr, h_out):
    c = cnt[0, :] + cnt[1, :]
    inv = 1.0 / jnp.maximum(c, 1.0)
    mean = (p0[...] + p1[...]) * inv[:, None]
    h = (jnp.dot(mean, wl[...], preferred_element_type=jnp.float32)
         + bl[...]
         + jnp.dot(x[...], wr[...], preferred_element_type=jnp.float32))
    h_out[...] = jnp.maximum(h, 0.0)


def _tc2_body(p0, p1, cnt, h, wl, bl, wr, wlin, blin, out):
    c = cnt[0, :] + cnt[1, :]
    inv = 1.0 / jnp.maximum(c, 1.0)
    mean = (p0[...] + p1[...]) * inv[:, None]
    h2 = (jnp.dot(mean, wl[...], preferred_element_type=jnp.float32)
          + bl[...]
          + jnp.dot(h[...], wr[...], preferred_element_type=jnp.float32))
    h2 = jnp.maximum(h2, 0.0)
    out[...] = (jnp.dot(h2, wlin[...], preferred_element_type=jnp.float32)
                + blin[...])


_row_spec = pl.BlockSpec((R, D), lambda i: (i, 0))
_cnt_spec = pl.BlockSpec((2, R), lambda i: (0, i))
_w_spec = pl.BlockSpec((D, D), lambda i: (0, 0))
_b_spec = pl.BlockSpec((1, D), lambda i: (0, 0))

_tc1 = pl.pallas_call(
    _tc1_body,
    grid=(GRID,),
    in_specs=[_row_spec, _row_spec, _cnt_spec, _row_spec,
              _w_spec, _b_spec, _w_spec],
    out_specs=_row_spec,
    out_shape=jax.ShapeDtypeStruct((NP, D), jnp.float32),
)

_tc2 = pl.pallas_call(
    _tc2_body,
    grid=(GRID,),
    in_specs=[_row_spec, _row_spec, _cnt_spec, _row_spec,
              _w_spec, _b_spec, _w_spec, _w_spec, _b_spec],
    out_specs=_row_spec,
    out_shape=jax.ShapeDtypeStruct((NP, D), jnp.float32),
)


def kernel(x, edge_index, Wl1, bl1, Wr1, Wl2, bl2, Wr2, Wlin, blin):
    src = edge_index[0].astype(jnp.int32)
    dst = edge_index[1].astype(jnp.int32)
    pad = EP - E
    src2d = jnp.concatenate(
        [src, jnp.zeros((pad,), jnp.int32)]).reshape(NCHUNK, CH)
    dst2d = jnp.concatenate(
        [dst, N + (jnp.arange(pad, dtype=jnp.int32) % (NP - N))]
    ).reshape(NCHUNK, CH)
    eidx = jnp.stack([src2d, dst2d], axis=1)

    x_pad = jnp.pad(x, ((0, NP - N), (0, 0)))
    zrows = jnp.zeros((RPT, D), jnp.float32)
    zcol = jnp.zeros((RPT,), jnp.float32)
    ones = jnp.ones((CH,), jnp.float32)

    parts1, counts = _sc_agg_counts(x_pad, eidx, zrows, zcol, ones)
    cnt2 = counts.reshape(2, NP)
    h = _tc1(parts1[:NP], parts1[NP:], cnt2, x_pad,
             Wl1, bl1.reshape(1, D), Wr1)

    parts2 = _sc_agg_nocounts(h, eidx, zrows, zcol, ones)
    out = _tc2(parts2[:NP], parts2[NP:], cnt2, h,
               Wl2, bl2.reshape(1, D), Wr2, Wlin, blin.reshape(1, D))
    return out[:N]


# speedup vs baseline: 1.2118x; 1.1410x over previous
import functools

import jax
import jax.numpy as jnp
from jax import lax
from jax.experimental import pallas as pl
from jax.experimental.pallas import tpu as pltpu
from jax.experimental.pallas import tpu_sc as plsc

N = 10000
D = 128
E = 320000

NP = 10240
CH = 128
NCHUNK = 2560
EP = NCHUNK * CH
NWORKERS = 32
CPW = NCHUNK // NWORKERS
RPT = NP // 16


NB = 2
NIB = 4


def _sc_aggregate(with_counts):
    if with_counts:
        out_type = [jax.ShapeDtypeStruct((2 * NP, D), jnp.float32),
                    jax.ShapeDtypeStruct((2 * NP,), jnp.float32)]
    else:
        out_type = jax.ShapeDtypeStruct((2 * NP, D), jnp.float32)

    scratch = [
        pltpu.VMEM_SHARED((NP, D), jnp.float32),
        pltpu.VMEM_SHARED((NP,), jnp.float32),
    ] + [pltpu.VMEM((CH, D), jnp.float32) for _ in range(NB)] \
      + [pltpu.VMEM((2, CH), jnp.int32) for _ in range(NIB)] + [
        pltpu.VMEM((CH,), jnp.float32),
    ] + [pltpu.SemaphoreType.DMA for _ in range(NIB + 3 * NB)]

    mesh = plsc.VectorSubcoreMesh(core_axis_name="c", subcore_axis_name="s")

    @functools.partial(pl.kernel, mesh=mesh, out_type=out_type,
                       scratch_types=scratch)
    def sc_agg(table_hbm, eidx_hbm, zrows_hbm, zcol_hbm, ones_hbm,
               *refs):
        if with_counts:
            parts_hbm, counts_hbm = refs[0], refs[1]
            scr = refs[2:]
        else:
            parts_hbm = refs[0]
            scr = refs[1:]
        acc_sh, cnt_sh = scr[0], scr[1]
        rows = scr[2:2 + NB]
        idxs = scr[2 + NB:2 + NB + NIB]
        ones_v = scr[2 + NB + NIB]
        p = 3 + NB + NIB
        isems = scr[p:p + NIB]
        gsems = scr[p + NIB:p + NIB + NB]
        rsems = scr[p + NIB + NB:p + NIB + 2 * NB]
        csems = scr[p + NIB + 2 * NB:p + NIB + 3 * NB]

        cid = lax.axis_index("c")
        sid = lax.axis_index("s")
        base = (cid * 16 + sid) * CPW
        sl = pl.ds(sid * RPT, RPT)

        pltpu.sync_copy(zrows_hbm, acc_sh.at[sl])
        if with_counts:
            pltpu.sync_copy(zcol_hbm, cnt_sh.at[sl])
        pltpu.sync_copy(ones_hbm, ones_v)
        plsc.subcore_barrier()

        def idx_cp(i, ib):
            return pltpu.make_async_copy(eidx_hbm.at[base + i], idxs[ib],
                                         isems[ib])

        def gather(r4, r2):
            return pltpu.make_async_copy(
                table_hbm.at[idxs[r4].at[0]], rows[r2], gsems[r2])

        def srow(r4, r2):
            return pltpu.make_async_copy(
                rows[r2], acc_sh.at[idxs[r4].at[1]], rsems[r2])

        def scnt(r4, r2):
            return pltpu.make_async_copy(
                ones_v, cnt_sh.at[idxs[r4].at[1]], csems[r2])

        idx_cp(0, 0).start()
        idx_cp(1, 1).start()
        idx_cp(2, 2).start()
        idx_cp(0, 0).wait()
        gather(0, 0).start()
        idx_cp(3, 3).start()
        idx_cp(1, 1).wait()
        gather(1, 1).start()
        gather(0, 0).wait()
        srow(0, 0).start(add=True)
        if with_counts:
            scnt(0, 0).start(add=True)

        def body(k, carry):
            i0 = 2 + 4 * k
            for j in range(4):
                i = i0 + j
                r4 = (2 + j) % 4
                r2 = j % 2
                p4 = (1 + j) % 4
                n4 = (j + 4) % 4
                srow(n4, r2).wait()
                if with_counts:
                    scnt(n4, r2).wait()
                idx_cp(i + 2, n4).start()
                idx_cp(i, r4).wait()
                gather(r4, r2).start()
                gather(p4, 1 - r2).wait()
                srow(p4, 1 - r2).start(add=True)
                if with_counts:
                    scnt(p4, 1 - r2).start(add=True)
            return carry

        lax.fori_loop(0, (CPW - 4) // 4, body, 0)

        for i in (CPW - 2, CPW - 1):
            r4, r2 = i % 4, i % 2
            p4 = (i - 1) % 4
            srow((i - 2) % 4, r2).wait()
            if with_counts:
                scnt((i - 2) % 4, r2).wait()
            idx_cp(i, r4).wait()
            gather(r4, r2).start()
            gather(p4, 1 - r2).wait()
            srow(p4, 1 - r2).start(add=True)
            if with_counts:
                scnt(p4, 1 - r2).start(add=True)
        lr4, lr2 = (CPW - 1) % 4, (CPW - 1) % 2
        gather(lr4, lr2).wait()
        srow(lr4, lr2).start(add=True)
        if with_counts:
            scnt(lr4, lr2).start(add=True)
        for i in (CPW - 2, CPW - 1):
            srow(i % 4, i % 2).wait()
            if with_counts:
                scnt(i % 4, i % 2).wait()
        plsc.subcore_barrier()

        off = cid * NP + sid * RPT
        pltpu.sync_copy(acc_sh.at[sl], parts_hbm.at[pl.ds(off, RPT)])
        if with_counts:
            pltpu.sync_copy(cnt_sh.at[sl], counts_hbm.at[pl.ds(off, RPT)])

    return sc_agg


_sc_agg_counts = _sc_aggregate(True)
_sc_agg_nocounts = _sc_aggregate(False)


R = 512
GRID = NP // R


def _tc1_body(p0, p1, cnt, x, wl, bl, wr, h_out):
    c = cnt[0, :] + cnt[1, :]
    inv = 1.0 / jnp.maximum(c, 1.0)
    mean = (p0[...] + p1[...]) * inv[:, None]
    h = (jnp.dot(mean, wl[...], preferred_element_type=jnp.float32)
         + bl[...]
         + jnp.dot(x[...], wr[...], preferred_element_type=jnp.float32))
    h_out[...] = jnp.maximum(h, 0.0)


def _tc2_body(p0, p1, cnt, h, wl, bl, wr, wlin, blin, out):
    c = cnt[0, :] + cnt[1, :]
    inv = 1.0 / jnp.maximum(c, 1.0)
    mean = (p0[...] + p1[...]) * inv[:, None]
    h2 = (jnp.dot(mean, wl[...], preferred_element_type=jnp.float32)
          + bl[...]
          + jnp.dot(h[...], wr[...], preferred_element_type=jnp.float32))
    h2 = jnp.maximum(h2, 0.0)
    out[...] = (jnp.dot(h2, wlin[...], preferred_element_type=jnp.float32)
                + blin[...])


_row_spec = pl.BlockSpec((R, D), lambda i: (i, 0))
_cnt_spec = pl.BlockSpec((2, R), lambda i: (0, i))
_w_spec = pl.BlockSpec((D, D), lambda i: (0, 0))
_b_spec = pl.BlockSpec((1, D), lambda i: (0, 0))

_tc1 = pl.pallas_call(
    _tc1_body,
    grid=(GRID,),
    in_specs=[_row_spec, _row_spec, _cnt_spec, _row_spec,
              _w_spec, _b_spec, _w_spec],
    out_specs=_row_spec,
    out_shape=jax.ShapeDtypeStruct((NP, D), jnp.float32),
)

_tc2 = pl.pallas_call(
    _tc2_body,
    grid=(GRID,),
    in_specs=[_row_spec, _row_spec, _cnt_spec, _row_spec,
              _w_spec, _b_spec, _w_spec, _w_spec, _b_spec],
    out_specs=_row_spec,
    out_shape=jax.ShapeDtypeStruct((NP, D), jnp.float32),
)


def kernel(x, edge_index, Wl1, bl1, Wr1, Wl2, bl2, Wr2, Wlin, blin):
    src = edge_index[0].astype(jnp.int32)
    dst = edge_index[1].astype(jnp.int32)
    pad = EP - E
    src2d = jnp.concatenate(
        [src, jnp.zeros((pad,), jnp.int32)]).reshape(NCHUNK, CH)
    dst2d = jnp.concatenate(
        [dst, N + (jnp.arange(pad, dtype=jnp.int32) % (NP - N))]
    ).reshape(NCHUNK, CH)
    eidx = jnp.stack([src2d, dst2d], axis=1)

    x_pad = jnp.pad(x, ((0, NP - N), (0, 0)))
    zrows = jnp.zeros((RPT, D), jnp.float32)
    zcol = jnp.zeros((RPT,), jnp.float32)
    ones = jnp.ones((CH,), jnp.float32)

    parts1, counts = _sc_agg_counts(x_pad, eidx, zrows, zcol, ones)
    cnt2 = counts.reshape(2, NP)
    h = _tc1(parts1[:NP], parts1[NP:], cnt2, x_pad,
             Wl1, bl1.reshape(1, D), Wr1)

    parts2 = _sc_agg_nocounts(h, eidx, zrows, zcol, ones)
    out = _tc2(parts2[:NP], parts2[NP:], cnt2, h,
               Wl2, bl2.reshape(1, D), Wr2, Wlin, blin.reshape(1, D))
    return out[:N]
